# Initial kernel scaffold; baseline (speedup 1.0000x reference)
import jax
import jax.numpy as jnp
from jax.experimental import pallas as pl


def kernel(x, W1, g1, beta1, W2, g2, beta2, W3, g3, beta3, W4, g4, beta4, W5, g5, beta5, Wemb, Wclu):
    raise NotImplementedError("write your pallas kernel here")


# speedup vs baseline: 4.6475x; 4.6475x over previous
import functools

import jax
import jax.numpy as jnp
from jax import lax
from jax.experimental import pallas as pl
from jax.experimental.pallas import tpu as pltpu
from jax.experimental.pallas import tpu_sc as plsc

KNN = 20
EPS = 1e-5
NEG_INF = float("-inf")


def _leaky(x):
    return jnp.where(x >= 0, x, 0.2 * x)


def _bf16(x):
    return x.astype(jnp.bfloat16)


def _stageA_body(N, RT, xf_ref, xr_ref, Wc_ref, idx_ref, w_ref):
    b = pl.program_id(0)
    xb = xf_ref[0]
    xr = xr_ref[0]
    G = lax.dot_general(_bf16(xr), _bf16(xb), (((1,), (1,)), ((), ())),
                        preferred_element_type=jnp.float32)
    xx = jnp.sum(xb * xb, axis=1)
    xxr = jnp.sum(xr * xr, axis=1)
    pd = 2.0 * G - xxr[:, None] - xx[None, :]
    iota = lax.broadcasted_iota(jnp.int32, (RT, N), 1)
    work = pd
    cols = []
    for _ in range(KNN):
        m = jnp.max(work, axis=1, keepdims=True)
        am = jnp.min(jnp.where(work == m, iota, N), axis=1, keepdims=True)
        cols.append(am)
        work = jnp.where(iota == am, NEG_INF, work)
    idx = jnp.concatenate(cols, axis=1)
    idx_ref[0] = idx + b * N
    w_ref[0] = lax.dot_general(_bf16(xr), _bf16(Wc_ref[...]),
                               (((1,), (1,)), ((), ())),
                               preferred_element_type=jnp.float32)


def _stageA(xr, Wc, B, N, RT):
    C = xr.shape[-1]
    O = Wc.shape[0]
    return pl.pallas_call(
        functools.partial(_stageA_body, N, RT),
        grid=(B, N // RT),
        in_specs=[
            pl.BlockSpec((1, N, C), lambda b, j: (b, 0, 0)),
            pl.BlockSpec((1, RT, C), lambda b, j: (b, j, 0)),
            pl.BlockSpec((O, C), lambda b, j: (0, 0)),
        ],
        out_specs=[
            pl.BlockSpec((1, RT, KNN), lambda b, j: (b, j, 0)),
            pl.BlockSpec((1, RT, O), lambda b, j: (b, j, 0)),
        ],
        out_shape=[
            jax.ShapeDtypeStruct((B, N, KNN), jnp.int32),
            jax.ShapeDtypeStruct((B, N, O), jnp.float32),
        ],
    )(xr, xr, Wc)


def _sc_gather_body(GP, NPTS, x_hbm, idx_hbm, out_hbm, idx_v, rows_v, sem):
    info = plsc.get_sparse_core_info()
    nc, ns = info.num_cores, info.num_subcores
    wid = lax.axis_index("s") * nc + lax.axis_index("c")
    nw = nc * ns
    ppw = NPTS // nw
    ng = ppw // GP
    pltpu.sync_copy(idx_hbm.at[pl.ds(wid * ng, ng)], idx_v)

    def group(g, _):
        pltpu.async_copy(x_hbm.at[idx_v.at[g]], rows_v, sem).wait()
        pltpu.sync_copy(rows_v, out_hbm.at[pl.ds((wid * ppw + g * GP) * KNN,
                                                 GP * KNN)])
        return 0

    lax.fori_loop(0, ng, group, 0)


def _sc_gather(x_rows, idx_flat, GP):
    NPTS, C = x_rows.shape
    nw = 32
    ng = (NPTS // nw) // GP
    idx2d = idx_flat.reshape(-1, GP * KNN)
    mesh = plsc.VectorSubcoreMesh(core_axis_name="c", subcore_axis_name="s")
    return pl.kernel(
        functools.partial(_sc_gather_body, GP, NPTS),
        out_type=jax.ShapeDtypeStruct((NPTS * KNN, C), jnp.float32),
        mesh=mesh,
        scratch_types=[
            pltpu.VMEM((ng, GP * KNN), jnp.int32),
            pltpu.VMEM((GP * KNN, C), jnp.float32),
            pltpu.SemaphoreType.DMA,
        ],
    )(x_rows, idx2d)


def _econv_body(PT, gath_ref, ctr_ref, Wn_ref, gmax_ref, gsum_ref, gsq_ref):
    C = gath_ref.shape[-1]
    ctr = ctr_ref[...]
    ctr3 = jnp.broadcast_to(ctr[:, None, :], (PT, KNN, C))
    diff = gath_ref[...] - ctr3.reshape(PT * KNN, C)
    un = lax.dot_general(_bf16(diff), _bf16(Wn_ref[...]),
                         (((1,), (1,)), ((), ())),
                         preferred_element_type=jnp.float32)
    O = un.shape[-1]
    r3 = un.reshape(PT, KNN, O)
    gmax_ref[...] = jnp.max(r3, axis=1)
    gsum_ref[...] = jnp.sum(r3, axis=1)
    gsq_ref[...] = jnp.sum(r3 * r3, axis=1)


def _econv(gath, x_rows, Wn, NPTS, PT):
    C = gath.shape[-1]
    O = Wn.shape[0]
    out_spec = pl.BlockSpec((PT, O), lambda t: (t, 0))
    out_t = jax.ShapeDtypeStruct((NPTS, O), jnp.float32)
    return pl.pallas_call(
        functools.partial(_econv_body, PT),
        grid=(NPTS // PT,),
        in_specs=[
            pl.BlockSpec((PT * KNN, C), lambda t: (t, 0)),
            pl.BlockSpec((PT, C), lambda t: (t, 0)),
            pl.BlockSpec((O, C), lambda t: (0, 0)),
        ],
        out_specs=[out_spec, out_spec, out_spec],
        out_shape=[out_t, out_t, out_t],
    )(gath, x_rows, Wn)


def _stats_body(M, gsum_ref, gsq_ref, v_ref, gb_ref, out_ref, acc):
    t = pl.program_id(0)
    gs = gsum_ref[...]
    gq = gsq_ref[...]
    v = v_ref[...]
    s1 = jnp.sum(gs, axis=0)
    s2 = jnp.sum(gq, axis=0)
    sv = jnp.sum(v, axis=0)
    svv = jnp.sum(v * v, axis=0)
    sx = jnp.sum(gs * v, axis=0)
    part = jnp.stack([s1, s2, sv, svv, sx], axis=0)

    @pl.when(t == 0)
    def _():
        acc[...] = part

    @pl.when(t > 0)
    def _():
        acc[...] = acc[...] + part

    @pl.when(t == pl.num_programs(0) - 1)
    def _():
        a = acc[...]
        inv_m = 1.0 / M
        mean = (a[0] + KNN * a[2]) * inv_m
        e2 = (a[1] + 2.0 * a[4] + KNN * a[3]) * inv_m
        var = e2 - mean * mean
        scale = gb_ref[0] / jnp.sqrt(var + EPS)
        shift = gb_ref[1] - mean * scale
        out_ref[...] = jnp.stack([scale, shift], axis=0)


def _stats(gsum, gsq, w_rows, g, beta, BT):
    NPTS, O = gsum.shape
    M = NPTS * KNN
    gb = jnp.stack([g, beta], axis=0)
    return pl.pallas_call(
        functools.partial(_stats_body, float(M)),
        grid=(NPTS // BT,),
        in_specs=[
            pl.BlockSpec((BT, O), lambda t: (t, 0)),
            pl.BlockSpec((BT, O), lambda t: (t, 0)),
            pl.BlockSpec((BT, O), lambda t: (t, 0)),
            pl.BlockSpec((2, O), lambda t: (0, 0)),
        ],
        out_specs=pl.BlockSpec((2, O), lambda t: (0, 0)),
        out_shape=jax.ShapeDtypeStruct((2, O), jnp.float32),
        scratch_shapes=[pltpu.VMEM((5, O), jnp.float32)],
    )(gsum, gsq, w_rows, gb)


def _norm_body(gmax_ref, v_ref, ss_ref, out_ref):
    y = gmax_ref[...] + v_ref[...]
    out_ref[...] = _leaky(y * ss_ref[0][None, :] + ss_ref[1][None, :])


def _norm(gmax, w_rows, ss, BT):
    NPTS, O = gmax.shape
    return pl.pallas_call(
        _norm_body,
        grid=(NPTS // BT,),
        in_specs=[
            pl.BlockSpec((BT, O), lambda t: (t, 0)),
            pl.BlockSpec((BT, O), lambda t: (t, 0)),
            pl.BlockSpec((2, O), lambda t: (0, 0)),
        ],
        out_specs=pl.BlockSpec((BT, O), lambda t: (t, 0)),
        out_shape=jax.ShapeDtypeStruct((NPTS, O), jnp.float32),
    )(gmax, w_rows, ss)


def _final_conv_body(o1, o2, x1_ref, x2_ref, x3_ref, x4_ref, W_ref,
                     ymax_ref, stats_ref, acc):
    t = pl.program_id(0)
    xcat = jnp.concatenate(
        [x1_ref[...][:, :o1], x2_ref[...][:, :o2], x3_ref[...], x4_ref[...]],
        axis=1)
    y = lax.dot_general(_bf16(xcat), _bf16(W_ref[...]),
                        (((1,), (1,)), ((), ())),
                        preferred_element_type=jnp.float32)
    s1 = jnp.sum(y, axis=0)
    s2 = jnp.sum(y * y, axis=0)
    part = jnp.stack([s1, s2], axis=0)

    @pl.when(t == 0)
    def _():
        acc[...] = part

    @pl.when(t > 0)
    def _():
        acc[...] = acc[...] + part

    ymax_ref[0] = jnp.max(y, axis=0, keepdims=True)

    @pl.when(t == pl.num_programs(0) - 1)
    def _():
        stats_ref[...] = acc[...]


def _final_conv(x1, x2, x3, x4, o1, o2, W5, B, N):
    O = W5.shape[0]
    return pl.pallas_call(
        functools.partial(_final_conv_body, o1, o2),
        grid=(B,),
        in_specs=[
            pl.BlockSpec((N, x1.shape[1]), lambda t: (t, 0)),
            pl.BlockSpec((N, x2.shape[1]), lambda t: (t, 0)),
            pl.BlockSpec((N, x3.shape[1]), lambda t: (t, 0)),
            pl.BlockSpec((N, x4.shape[1]), lambda t: (t, 0)),
            pl.BlockSpec((O, O), lambda t: (0, 0)),
        ],
        out_specs=[
            pl.BlockSpec((1, 1, O), lambda t: (t, 0, 0)),
            pl.BlockSpec((2, O), lambda t: (0, 0)),
        ],
        out_shape=[
            jax.ShapeDtypeStruct((B, 1, O), jnp.float32),
            jax.ShapeDtypeStruct((2, O), jnp.float32),
        ],
        scratch_shapes=[pltpu.VMEM((2, O), jnp.float32)],
    )(x1, x2, x3, x4, W5)


def _head_body(M, ymax_ref, stats_ref, gb_ref, Wemb_ref, Wclu_ref,
               x0_ref, emb_ref, q_ref):
    inv_m = 1.0 / M
    st = stats_ref[...]
    mean = st[0] * inv_m
    var = st[1] * inv_m - mean * mean
    scale = gb_ref[0] / jnp.sqrt(var + EPS)
    shift = gb_ref[1] - mean * scale
    x0 = _leaky(ymax_ref[...] * scale[None, :] + shift[None, :])
    x0_ref[...] = x0
    emb = lax.dot_general(_bf16(x0), _bf16(Wemb_ref[...]),
                          (((1,), (1,)), ((), ())),
                          preferred_element_type=jnp.float32)
    emb_ref[...] = emb
    d = emb[:, None, :] - Wclu_ref[...][None, :, :]
    s = jnp.sum(d * d, axis=2)
    q = 1.0 / (1.0 + s)
    q_ref[...] = q / jnp.sum(q, axis=1, keepdims=True)


def _head(ymax, stats, g5, b5, Wemb, Wclu, B, N):
    gb = jnp.stack([g5, b5], axis=0)
    return pl.pallas_call(
        functools.partial(_head_body, float(B * N)),
        out_shape=[
            jax.ShapeDtypeStruct((B, 512), jnp.float32),
            jax.ShapeDtypeStruct((B, Wemb.shape[0]), jnp.float32),
            jax.ShapeDtypeStruct((B, Wclu.shape[0]), jnp.float32),
        ],
    )(ymax, stats, gb, Wemb, Wclu)


def _pad_to(a, axis, size):
    pad = size - a.shape[axis]
    if pad == 0:
        return a
    cfg = [(0, 0)] * a.ndim
    cfg[axis] = (0, pad)
    return jnp.pad(a, cfg)


def _edgeconv_stage(xr, xg, W, g, beta, B, N, C_true):
    O = W.shape[0]
    C_pad = xr.shape[-1]
    C_g = xg.shape[-1]
    O_pad = max(128, O)
    Wn = _pad_to(_pad_to(W[:, :C_true], 1, C_g), 0, O_pad)
    Wc = _pad_to(_pad_to(W[:, C_true:], 1, C_pad), 0, O_pad)
    g_pad = _pad_to(g, 0, O_pad) + jnp.pad(
        jnp.zeros((O,), jnp.float32), (0, O_pad - O), constant_values=1.0)
    beta_pad = _pad_to(beta, 0, O_pad)

    GP = 4
    idx, w = _stageA(xr, Wc, B, N, RT=256)
    w_rows = w.reshape(B * N, O_pad)
    gath = _sc_gather(xg, idx.reshape(-1), GP)
    gmax, gsum, gsq = _econv(gath, xg, Wn, B * N, PT=256)
    ss = _stats(gsum, gsq, w_rows, g_pad, beta_pad, BT=1024)
    return _norm(gmax, w_rows, ss, BT=1024)


def kernel(x, W1, g1, beta1, W2, g2, beta2, W3, g3, beta3, W4, g4, beta4,
           W5, g5, beta5, Wemb, Wclu):
    B, N, C0 = x.shape
    NP = B * N
    x0 = _pad_to(x, 2, 16)
    xg0 = _pad_to(x, 2, 128).reshape(NP, 128)

    x1 = _edgeconv_stage(x0, xg0, W1, g1, beta1, B, N, C_true=C0)
    x2 = _edgeconv_stage(x1.reshape(B, N, -1), x1, W2, g2, beta2,
                         B, N, C_true=64)
    x3 = _edgeconv_stage(x2.reshape(B, N, -1), x2, W3, g3, beta3,
                         B, N, C_true=64)
    x4 = _edgeconv_stage(x3.reshape(B, N, -1), x3, W4, g4, beta4,
                         B, N, C_true=128)

    ymax, stats = _final_conv(x1, x2, x3, x4, 64, 64, W5, B, N)
    x0max, emb, q = _head(ymax.reshape(B, 512), stats, g5, beta5, Wemb, Wclu,
                          B, N)
    feat = x0max[:, None, :]
    return feat, emb, q
